# baseline (device time: 20889 ns/iter reference)
import jax
import jax.numpy as jnp
from jax import lax
from jax.experimental import pallas as pl
from jax.experimental.pallas import tpu as pltpu

N_DEV = 4


def kernel(x, k, Wp):
    b, s, c = x.shape
    n = Wp.shape[1]
    bs = b * s
    h = bs // 2

    def body(x_ref, k_ref, wp_ref, out_ref,
             p1s, p1r, p2s, p2r,
             send_sems, recv_sems):
        my = lax.axis_index("i")
        pa = my ^ 1
        pb = 3 - my

        barrier_sem = pltpu.get_barrier_semaphore()
        for nbr in (pa, pb):
            pl.semaphore_signal(
                barrier_sem, inc=1,
                device_id=(nbr,), device_id_type=pl.DeviceIdType.MESH,
            )

        kv = k_ref[...].astype(jnp.bfloat16)
        wv = wp_ref[...].astype(jnp.bfloat16)

        def chunk_partial(i):
            xv = x_ref[i].astype(jnp.bfloat16)
            acc = xv * kv[3][None, :]
            for shift in range(1, 4):
                shifted = jnp.concatenate(
                    [jnp.zeros((shift, c), xv.dtype), xv[: s - shift, :]],
                    axis=0,
                )
                acc = acc + shifted * kv[3 - shift][None, :]
            a = acc * jax.nn.sigmoid(acc)
            return lax.dot_general(
                a, wv,
                dimension_numbers=(((1,), (0,)), ((), ())),
                preferred_element_type=jnp.float32,
            ).astype(jnp.bfloat16)

        p1_dst = {0: pa, 1: pa, 2: pb, 3: pb}
        p2_dst = {0: pb, 1: pb, 2: pa, 3: pa}
        ex1 = {
            i: pltpu.make_async_remote_copy(
                src_ref=p1s.at[i], dst_ref=p1r.at[i],
                send_sem=send_sems.at[i], recv_sem=recv_sems.at[i],
                device_id=(p1_dst[i],), device_id_type=pl.DeviceIdType.MESH,
            )
            for i in range(4)
        }
        ex2 = {
            i: pltpu.make_async_remote_copy(
                src_ref=p2s.at[i], dst_ref=p2r.at[i],
                send_sem=send_sems.at[4 + i], recv_sem=recv_sems.at[4 + i],
                device_id=(p2_dst[i],), device_id_type=pl.DeviceIdType.MESH,
            )
            for i in range(4)
        }

        order = (0, 2, 1, 3)
        p1s[order[0]] = chunk_partial(order[0])
        pl.semaphore_wait(barrier_sem, 2)
        ex1[order[0]].start()
        for i in order[1:]:
            p1s[i] = chunk_partial(i)
            ex1[i].start()

        for i in order:
            ex1[i].wait()
            p2s[i] = p1s[i] + p1r[i]
            ex2[i].start()

        for i in order:
            ex2[i].wait()
            out_ref[i] = (p2s[i] + p2r[i]).reshape(s, n)

    half = (h, n)
    return pl.pallas_call(
        body,
        out_shape=jax.ShapeDtypeStruct((b, s, n), jnp.bfloat16),
        in_specs=[pl.BlockSpec(memory_space=pltpu.VMEM)] * 3,
        out_specs=pl.BlockSpec(memory_space=pltpu.VMEM),
        scratch_shapes=[
            pltpu.VMEM((4, s, n), jnp.bfloat16),
            pltpu.VMEM((4, s, n), jnp.bfloat16),
            pltpu.VMEM((4, s, n), jnp.bfloat16),
            pltpu.VMEM((4, s, n), jnp.bfloat16),
            pltpu.SemaphoreType.DMA((8,)),
            pltpu.SemaphoreType.DMA((8,)),
        ],
        compiler_params=pltpu.CompilerParams(collective_id=0),
    )(x, k, Wp)
